# baseline (speedup 1.0000x reference)
import functools

import jax
import jax.numpy as jnp
from jax import lax
from jax.experimental import pallas as pl
from jax.experimental.pallas import tpu as pltpu
from jax.experimental.pallas import tpu_sc as plsc

V = 1_000_000
D = 64
VP = V // 2
NC, NS = 2, 16
NW = NC * NS
B = 4096 * 200
NBLK = B // 128
BLK_PER_W = NBLK // NW
FULL_COLS = (V // 128) * 128
NFULL2 = FULL_COLS // 256
RPW = 120
NPEEL = 2
NEXTRA = NFULL2 - (RPW + NPEEL) * NW


def _iota16():
    return lax.iota(jnp.int32, 16)


def _repack_block(in_v, out_v, nrows):
    rot = _iota16()
    r_vec = lax.shift_right_logical(rot, 1)
    h64_vec = lax.bitwise_and(rot, 1) * 64

    for q in range(nrows // 8):
        col_vec = rot + 16 * q
        r_q = r_vec + 8 * q

        @plsc.parallel_loop(0, D, unroll=8)
        def dim(d):
            d_vec = lax.bitwise_and(rot + d, 63)
            vals = plsc.load_gather(in_v, [d_vec, col_vec])
            plsc.store_scatter(out_v, [r_q, d_vec + h64_vec], vals)


def _build_repack():
    mesh = plsc.VectorSubcoreMesh(core_axis_name="c", subcore_axis_name="s")

    @functools.partial(
        pl.kernel,
        mesh=mesh,
        out_type=jax.ShapeDtypeStruct((VP, 128), jnp.float32),
        scratch_types=[
            pltpu.VMEM((D, 256), jnp.float32),
            pltpu.VMEM((D, 256), jnp.float32),
            pltpu.VMEM((D, 256), jnp.float32),
            pltpu.VMEM((D, 64), jnp.float32),
            pltpu.VMEM((128, 128), jnp.float32),
            pltpu.VMEM((128, 128), jnp.float32),
            pltpu.VMEM((128, 128), jnp.float32),
            pltpu.SemaphoreType.DMA,
            pltpu.SemaphoreType.DMA,
            pltpu.SemaphoreType.DMA,
            pltpu.SemaphoreType.DMA,
            pltpu.SemaphoreType.DMA,
            pltpu.SemaphoreType.DMA,
        ],
        compiler_params=pltpu.CompilerParams(use_tc_tiling_on_sc=True, needs_layout_passes=False),
    )
    def repack(
        tT_hbm, r_hbm, in0, in1, in2, int_, o0, o1, o2,
        si0, si1, si2, so0, so1, so2,
    ):
        wid = lax.axis_index("s") * NC + lax.axis_index("c")
        m0 = wid * RPW

        def in_desc(m, buf, sem):
            return pltpu.make_async_copy(
                tT_hbm.at[:, pl.ds(m * 256, 256)], buf, sem
            )

        def out_desc(m, buf, sem):
            return pltpu.make_async_copy(
                buf, r_hbm.at[pl.ds(m * 128, 128), :], sem
            )

        bufs = (
            (0, in0, o0, si0, so0),
            (1, in1, o1, si1, so1),
            (2, in2, o2, si2, so2),
        )
        for (k, in_v, _, si, _2) in bufs:
            in_desc(m0 + k, in_v, si).start()

        def body(u, carry):
            ma = m0 + 3 * u
            for (k, in_v, out_v, si, so) in bufs:
                mb = ma + k
                in_desc(mb, in_v, si).wait()
                pl.when(u > 0)(lambda: out_desc(mb - 3, out_v, so).wait())
                _repack_block(in_v, out_v, 128)
                out_desc(mb, out_v, so).start()
                pl.when(u < RPW // 3 - 1)(
                    lambda: in_desc(mb + 3, in_v, si).start()
                )
            return carry

        lax.fori_loop(0, RPW // 3, body, 0)
        for (k, _, out_v, _2, so) in bufs:
            out_desc(m0 + RPW - 3 + k, out_v, so).wait()

        def peel(m):
            in_desc(m, in0, si0).start()
            in_desc(m, in0, si0).wait()
            _repack_block(in0, o0, 128)
            out_desc(m, o0, so0).start()
            out_desc(m, o0, so0).wait()

        base_p = RPW * NW + wid * NPEEL
        for p in range(NPEEL):
            peel(base_p + p)

        @pl.when(wid < NEXTRA)
        def _extra():
            peel(NFULL2 - NEXTRA + wid)

        @pl.when(wid == NW - 1)
        def _tail():
            tin = pltpu.make_async_copy(
                tT_hbm.at[:, pl.ds(FULL_COLS, 64)], int_, si1
            )
            tin.start()
            tin.wait()
            _repack_block(int_, o1, 32)
            tout = pltpu.make_async_copy(
                o1.at[pl.ds(0, 32), :],
                r_hbm.at[pl.ds(FULL_COLS // 2, 32), :],
                so1,
            )
            tout.start()
            tout.wait()

    return repack


def _build_gather():
    mesh = plsc.VectorSubcoreMesh(core_axis_name="c", subcore_axis_name="s")

    @functools.partial(
        pl.kernel,
        mesh=mesh,
        out_type=jax.ShapeDtypeStruct((200, 8, 32, 8, 128), jnp.float32),
        scratch_types=[
            pltpu.VMEM((BLK_PER_W * 128,), jnp.int32),
            pltpu.VMEM((128, D), jnp.float32),
            pltpu.VMEM((128, D), jnp.float32),
            pltpu.VMEM((128, D), jnp.float32),
            pltpu.VMEM((128, D), jnp.float32),
            pltpu.VMEM((8, 8, 128), jnp.float32),
            pltpu.VMEM((8, 8, 128), jnp.float32),
            pltpu.VMEM((8, 8, 128), jnp.float32),
            pltpu.VMEM((8, 8, 128), jnp.float32),
            pltpu.SemaphoreType.DMA,
            pltpu.SemaphoreType.DMA,
            pltpu.SemaphoreType.DMA,
            pltpu.SemaphoreType.DMA,
            pltpu.SemaphoreType.DMA,
            pltpu.SemaphoreType.DMA,
            pltpu.SemaphoreType.DMA,
            pltpu.SemaphoreType.DMA,
            pltpu.SemaphoreType.DMA,
        ],
        compiler_params=pltpu.CompilerParams(use_tc_tiling_on_sc=False, needs_layout_passes=False),
    )
    def gather(
        xf_hbm, r_hbm, out_hbm,
        ixall, rw0, rw1, rw2, rw3, sl0, sl1, sl2, sl3,
        sxa, sg0, sg1, sg2, sg3, so0, so1, so2, so3,
    ):
        wid = lax.axis_index("s") * NC + lax.axis_index("c")
        b0 = wid * BLK_PER_W

        def gat_desc(t, buf, sem):
            return pltpu.make_async_copy(
                r_hbm.at[ixall.at[pl.ds(t * 128, 128)]], buf, sem
            )

        def out_desc(b, buf, sem):
            j = lax.shift_right_logical(b, 5)
            i_hi = lax.bitwise_and(b, 31)
            return pltpu.make_async_copy(
                buf, out_hbm.at[j, :, i_hi, :, :], sem
            )

        def transpose(rw, sl):
            i_vecs = [_iota16() + 16 * g for g in range(8)]
            rot = _iota16()

            @plsc.parallel_loop(0, D, unroll=8)
            def row(d):
                d_vec = lax.bitwise_and(rot + d, 63)
                d_hi = lax.shift_right_logical(d_vec, 3)
                d_lo = lax.bitwise_and(d_vec, 7)
                for g in range(8):
                    vals = plsc.load_gather(rw, [i_vecs[g], d_vec])
                    plsc.store_scatter(sl, [d_hi, d_lo, i_vecs[g]], vals)

        ixa_desc = pltpu.make_async_copy(
            xf_hbm.at[pl.ds(b0 * 128, BLK_PER_W * 128)], ixall, sxa
        )
        ixa_desc.start()
        ixa_desc.wait()
        bufs = (
            (0, rw0, sl0, sg0, so0),
            (1, rw1, sl1, sg1, so1),
            (2, rw2, sl2, sg2, so2),
            (3, rw3, sl3, sg3, so3),
        )
        for (k, rw, _, sg, _2) in bufs:
            gat_desc(k, rw, sg).start()

        def body(u, carry):
            for (k, rw, sl, sg, so) in bufs:
                t = 4 * u + k
                bb = b0 + t
                gat_desc(t, rw, sg).wait()
                pl.when(u > 0)(lambda: out_desc(bb - 4, sl, so).wait())
                transpose(rw, sl)
                out_desc(bb, sl, so).start()
                pl.when(u < BLK_PER_W // 4 - 1)(
                    lambda: gat_desc(t + 4, rw, sg).start()
                )

            return carry

        lax.fori_loop(0, BLK_PER_W // 4, body, 0)
        for (k, _, sl, _2, so) in bufs:
            out_desc(b0 + BLK_PER_W - 4 + k, sl, so).wait()

    return gather


def kernel(x, table):
    xf = x.T.reshape(B).astype(jnp.int32)
    tT = table.T
    repacked = _build_repack()(tT)
    r_lin = repacked.reshape(V, D)
    out5 = _build_gather()(xf, r_lin)
    return out5.transpose(2, 4, 0, 1, 3).reshape(4096, 200, D)


# speedup vs baseline: 1.1572x; 1.1572x over previous
import functools

import jax
import jax.numpy as jnp
from jax import lax
from jax.experimental import pallas as pl
from jax.experimental.pallas import tpu as pltpu
from jax.experimental.pallas import tpu_sc as plsc

V = 1_000_000
D = 64
VP = V // 2
NC, NS = 2, 16
NW = NC * NS
B = 4096 * 200
NBLK = B // 128
BLK_PER_W = NBLK // NW
FULL_COLS = (V // 128) * 128
NFULL2 = FULL_COLS // 256
RPW = 120
NPEEL = 2
NEXTRA = NFULL2 - (RPW + NPEEL) * NW


def _iota16():
    return lax.iota(jnp.int32, 16)


def _repack_block(in_v, out_v, nrows):
    rot = _iota16()
    r_vec = lax.shift_right_logical(rot, 1)
    h64_vec = lax.bitwise_and(rot, 1) * 64

    for qq in range(nrows // 16):
        cols = [rot + 16 * (2 * qq + t) for t in range(2)]
        rqs = [r_vec + 8 * (2 * qq + t) for t in range(2)]

        @plsc.parallel_loop(0, D, unroll=8)
        def dim(d):
            d_vec = lax.bitwise_and(rot + d, 63)
            dh = d_vec + h64_vec
            for t in range(2):
                vals = plsc.load_gather(in_v, [d_vec, cols[t]])
                plsc.store_scatter(out_v, [rqs[t], dh], vals)


def _build_repack():
    mesh = plsc.VectorSubcoreMesh(core_axis_name="c", subcore_axis_name="s")

    @functools.partial(
        pl.kernel,
        mesh=mesh,
        out_type=jax.ShapeDtypeStruct((VP, 128), jnp.float32),
        scratch_types=[
            pltpu.VMEM((D, 256), jnp.float32),
            pltpu.VMEM((D, 256), jnp.float32),
            pltpu.VMEM((D, 256), jnp.float32),
            pltpu.VMEM((D, 64), jnp.float32),
            pltpu.VMEM((128, 128), jnp.float32),
            pltpu.VMEM((128, 128), jnp.float32),
            pltpu.VMEM((128, 128), jnp.float32),
            pltpu.SemaphoreType.DMA,
            pltpu.SemaphoreType.DMA,
            pltpu.SemaphoreType.DMA,
            pltpu.SemaphoreType.DMA,
            pltpu.SemaphoreType.DMA,
            pltpu.SemaphoreType.DMA,
        ],
        compiler_params=pltpu.CompilerParams(use_tc_tiling_on_sc=True, needs_layout_passes=False),
    )
    def repack(
        tT_hbm, r_hbm, in0, in1, in2, int_, o0, o1, o2,
        si0, si1, si2, so0, so1, so2,
    ):
        wid = lax.axis_index("s") * NC + lax.axis_index("c")
        m0 = wid * RPW

        def in_desc(m, buf, sem):
            return pltpu.make_async_copy(
                tT_hbm.at[:, pl.ds(m * 256, 256)], buf, sem
            )

        def out_desc(m, buf, sem):
            return pltpu.make_async_copy(
                buf, r_hbm.at[pl.ds(m * 128, 128), :], sem
            )

        bufs = (
            (0, in0, o0, si0, so0),
            (1, in1, o1, si1, so1),
            (2, in2, o2, si2, so2),
        )
        for (k, in_v, _, si, _2) in bufs:
            in_desc(m0 + k, in_v, si).start()

        def body(u, carry):
            ma = m0 + 3 * u
            for (k, in_v, out_v, si, so) in bufs:
                mb = ma + k
                in_desc(mb, in_v, si).wait()
                pl.when(u > 0)(lambda: out_desc(mb - 3, out_v, so).wait())
                _repack_block(in_v, out_v, 128)
                out_desc(mb, out_v, so).start()
                pl.when(u < RPW // 3 - 1)(
                    lambda: in_desc(mb + 3, in_v, si).start()
                )
            return carry

        lax.fori_loop(0, RPW // 3, body, 0)
        for (k, _, out_v, _2, so) in bufs:
            out_desc(m0 + RPW - 3 + k, out_v, so).wait()

        def peel(m):
            in_desc(m, in0, si0).start()
            in_desc(m, in0, si0).wait()
            _repack_block(in0, o0, 128)
            out_desc(m, o0, so0).start()
            out_desc(m, o0, so0).wait()

        base_p = RPW * NW + wid * NPEEL
        for p in range(NPEEL):
            peel(base_p + p)

        @pl.when(wid < NEXTRA)
        def _extra():
            peel(NFULL2 - NEXTRA + wid)

        @pl.when(wid == NW - 1)
        def _tail():
            tin = pltpu.make_async_copy(
                tT_hbm.at[:, pl.ds(FULL_COLS, 64)], int_, si1
            )
            tin.start()
            tin.wait()
            _repack_block(int_, o1, 32)
            tout = pltpu.make_async_copy(
                o1.at[pl.ds(0, 32), :],
                r_hbm.at[pl.ds(FULL_COLS // 2, 32), :],
                so1,
            )
            tout.start()
            tout.wait()

    return repack


def _build_gather():
    mesh = plsc.VectorSubcoreMesh(core_axis_name="c", subcore_axis_name="s")

    @functools.partial(
        pl.kernel,
        mesh=mesh,
        out_type=jax.ShapeDtypeStruct((200, 8, 32, 8, 128), jnp.float32),
        scratch_types=[
            pltpu.VMEM((BLK_PER_W * 128,), jnp.int32),
            pltpu.VMEM((128, D), jnp.float32),
            pltpu.VMEM((128, D), jnp.float32),
            pltpu.VMEM((128, D), jnp.float32),
            pltpu.VMEM((128, D), jnp.float32),
            pltpu.VMEM((8, 8, 128), jnp.float32),
            pltpu.VMEM((8, 8, 128), jnp.float32),
            pltpu.VMEM((8, 8, 128), jnp.float32),
            pltpu.VMEM((8, 8, 128), jnp.float32),
            pltpu.SemaphoreType.DMA,
            pltpu.SemaphoreType.DMA,
            pltpu.SemaphoreType.DMA,
            pltpu.SemaphoreType.DMA,
            pltpu.SemaphoreType.DMA,
            pltpu.SemaphoreType.DMA,
            pltpu.SemaphoreType.DMA,
            pltpu.SemaphoreType.DMA,
            pltpu.SemaphoreType.DMA,
        ],
        compiler_params=pltpu.CompilerParams(use_tc_tiling_on_sc=False, needs_layout_passes=False),
    )
    def gather(
        xf_hbm, r_hbm, out_hbm,
        ixall, rw0, rw1, rw2, rw3, sl0, sl1, sl2, sl3,
        sxa, sg0, sg1, sg2, sg3, so0, so1, so2, so3,
    ):
        wid = lax.axis_index("s") * NC + lax.axis_index("c")
        b0 = wid * BLK_PER_W

        def gat_desc(t, buf, sem):
            return pltpu.make_async_copy(
                r_hbm.at[ixall.at[pl.ds(t * 128, 128)]], buf, sem
            )

        def out_desc(b, buf, sem):
            j = lax.shift_right_logical(b, 5)
            i_hi = lax.bitwise_and(b, 31)
            return pltpu.make_async_copy(
                buf, out_hbm.at[j, :, i_hi, :, :], sem
            )

        def transpose(rw, sl):
            i_vecs = [_iota16() + 16 * g for g in range(8)]
            rot = _iota16()

            @plsc.parallel_loop(0, D, unroll=8)
            def row(d):
                d_vec = lax.bitwise_and(rot + d, 63)
                d_hi = lax.shift_right_logical(d_vec, 3)
                d_lo = lax.bitwise_and(d_vec, 7)
                for g in range(8):
                    vals = plsc.load_gather(rw, [i_vecs[g], d_vec])
                    plsc.store_scatter(sl, [d_hi, d_lo, i_vecs[g]], vals)

        ixa_desc = pltpu.make_async_copy(
            xf_hbm.at[pl.ds(b0 * 128, BLK_PER_W * 128)], ixall, sxa
        )
        ixa_desc.start()
        ixa_desc.wait()
        bufs = (
            (0, rw0, sl0, sg0, so0),
            (1, rw1, sl1, sg1, so1),
            (2, rw2, sl2, sg2, so2),
            (3, rw3, sl3, sg3, so3),
        )
        for (k, rw, _, sg, _2) in bufs:
            gat_desc(k, rw, sg).start()

        def body(u, carry):
            for (k, rw, sl, sg, so) in bufs:
                t = 4 * u + k
                bb = b0 + t
                gat_desc(t, rw, sg).wait()
                pl.when(u > 0)(lambda: out_desc(bb - 4, sl, so).wait())
                transpose(rw, sl)
                out_desc(bb, sl, so).start()
                pl.when(u < BLK_PER_W // 4 - 1)(
                    lambda: gat_desc(t + 4, rw, sg).start()
                )

            return carry

        lax.fori_loop(0, BLK_PER_W // 4, body, 0)
        for (k, _, sl, _2, so) in bufs:
            out_desc(b0 + BLK_PER_W - 4 + k, sl, so).wait()

    return gather


def kernel(x, table):
    xf = x.T.reshape(B).astype(jnp.int32)
    tT = table.T
    repacked = _build_repack()(tT)
    r_lin = repacked.reshape(V, D)
    out5 = _build_gather()(xf, r_lin)
    return out5.transpose(2, 4, 0, 1, 3).reshape(4096, 200, D)
